# Initial kernel scaffold; baseline (speedup 1.0000x reference)
import jax
import jax.numpy as jnp
from jax.experimental import pallas as pl


def kernel(features, rois):
    raise NotImplementedError("write your pallas kernel here")


# speedup vs baseline: 4.8002x; 4.8002x over previous
import jax
import jax.numpy as jnp
from jax.experimental import pallas as pl
from jax.experimental.pallas import tpu as pltpu

OUT_H = 7
OUT_W = 7
WIN = 6
K = 8


def _roi_pool_body(params_ref, f_ref, out_ref, tmp_ref):
    neg = jnp.float32(-jnp.inf)
    for k in range(K):
        b = params_ref[k, 0]
        cols = []
        for wb in range(OUT_W):
            s = params_ref[k, 1 + wb]
            e = params_ref[k, 8 + wb]
            s0 = jnp.minimum(s, 32 - WIN)
            win = f_ref[b, pl.ds(s0, WIN)]
            idx = s0 + jax.lax.broadcasted_iota(jnp.int32, (WIN, 1, 1), 0)
            m = (idx >= s) & (idx < e)
            cols.append(jnp.max(jnp.where(m, win, neg), axis=0))
        tmp_ref[...] = jnp.stack(cols, axis=1)
        rows = []
        for hb in range(OUT_H):
            s = params_ref[k, 15 + hb]
            e = params_ref[k, 22 + hb]
            s0 = jnp.minimum(s, 32 - WIN)
            win = tmp_ref[pl.ds(s0, WIN)]
            idx = s0 + jax.lax.broadcasted_iota(jnp.int32, (WIN, 1, 1), 0)
            m = (idx >= s) & (idx < e)
            rows.append(jnp.max(jnp.where(m, win, neg), axis=0))
        pooled = jnp.stack(rows, axis=0)
        pooled = jnp.where(pooled > neg, pooled, jnp.float32(0.0))
        out_ref[k] = pooled.reshape(OUT_H * OUT_W, 96)


def _bin_params(rois):
    rois_i = rois.astype(jnp.int32)
    start_w = rois_i[:, 1].astype(jnp.float32)
    start_h = rois_i[:, 2].astype(jnp.float32)
    end_w = rois_i[:, 3].astype(jnp.float32)
    end_h = rois_i[:, 4].astype(jnp.float32)
    bin_h = jnp.maximum(end_h - start_h, 1.0) / float(OUT_H)
    bin_w = jnp.maximum(end_w - start_w, 1.0) / float(OUT_W)
    hs = jnp.arange(OUT_H, dtype=jnp.float32)
    ws = jnp.arange(OUT_W, dtype=jnp.float32)
    h_start = jnp.floor(hs[None, :] * bin_h[:, None] + start_h[:, None])
    h_end = jnp.ceil((hs[None, :] + 1.0) * bin_h[:, None] + start_h[:, None])
    w_start = jnp.floor(ws[None, :] * bin_w[:, None] + start_w[:, None])
    w_end = jnp.ceil((ws[None, :] + 1.0) * bin_w[:, None] + start_w[:, None])
    clip = lambda a: jnp.clip(a, 0, 32).astype(jnp.int32)
    return jnp.concatenate(
        [
            rois_i[:, :1],
            clip(w_start), clip(w_end), clip(h_start), clip(h_end),
            jnp.zeros((rois_i.shape[0], 3), jnp.int32),
        ],
        axis=1,
    )


def kernel(features, rois):
    N = rois.shape[0]
    C = features.shape[1]
    params = _bin_params(rois)
    fT = jnp.transpose(features, (0, 3, 2, 1))
    out = pl.pallas_call(
        _roi_pool_body,
        grid=(N // K,),
        in_specs=[
            pl.BlockSpec((K, 32), lambda i: (i, 0), memory_space=pltpu.SMEM),
            pl.BlockSpec((32, 32, 32, C), lambda i: (0, 0, 0, 0)),
        ],
        out_specs=pl.BlockSpec((K, OUT_H * OUT_W, C), lambda i: (i, 0, 0)),
        out_shape=jax.ShapeDtypeStruct((N, OUT_H * OUT_W, C), jnp.float32),
        scratch_shapes=[pltpu.VMEM((32, OUT_W, C), jnp.float32)],
    )(params, fT)
    return jnp.transpose(out.reshape(N, OUT_H, OUT_W, C), (0, 3, 1, 2))
